# baseline (speedup 1.0000x reference)
import jax
import jax.numpy as jnp
from jax.experimental import pallas as pl
from jax.experimental.pallas import tpu as pltpu

_BATCH = 32
_BEAM = 8
_VOCAB = 100000
_EOS = 2
_NEG = -3.0e38


def _beam_block(pred_ref, scores_ref, eos_ref, out_s_ref, out_t_ref, out_e_ref,
                x_ref):
    x_ref[...] = pred_ref[...]
    col = jax.lax.broadcasted_iota(jnp.int32, (_BEAM, _VOCAB), 1)

    vals = []
    idxs = []
    for _ in range(_BEAM):
        x = x_ref[...]
        m = jnp.max(x, axis=1, keepdims=True)
        eq = x == m
        idx = jnp.min(jnp.where(eq, col, jnp.int32(2**30)), axis=1,
                      keepdims=True)
        x_ref[...] = jnp.where(col == idx, _NEG, x)
        vals.append(m)
        idxs.append(idx)
    top_v = jnp.concatenate(vals, axis=1)
    top_i = jnp.concatenate(idxs, axis=1)

    beam_s = scores_ref[0, 0, :].reshape(_BEAM, 1)
    eos = eos_ref[0, 0, :].reshape(_BEAM, 1)
    cand = beam_s + jnp.log(top_v) * eos
    row_i = jax.lax.broadcasted_iota(jnp.int32, (_BEAM, _BEAM), 0)
    col_i = jax.lax.broadcasted_iota(jnp.int32, (_BEAM, _BEAM), 1)
    flat = row_i * _BEAM + col_i
    pcol = jax.lax.broadcasted_iota(jnp.int32, (1, _BEAM), 1)
    eos_col = jax.lax.broadcasted_iota(jnp.int32, (_BEAM, 1), 0)

    o_s = jnp.zeros((1, _BEAM), jnp.float32)
    o_t = jnp.zeros((1, _BEAM), jnp.int32)
    o_e = jnp.zeros((1, _BEAM), jnp.float32)
    for j in range(_BEAM):
        m = jnp.max(cand)
        eq = cand == m
        sel = jnp.min(jnp.where(eq, flat, jnp.int32(2**30)))
        hit = flat == sel
        tok = jnp.sum(jnp.where(hit, top_i, 0))
        parent = sel // _BEAM
        pe = jnp.sum(jnp.where(eos_col == parent, eos, jnp.float32(0.0)))
        slot = pcol == j
        o_s = jnp.where(slot, m, o_s)
        o_t = jnp.where(slot, tok, o_t)
        o_e = jnp.where(slot, pe * (tok != _EOS).astype(jnp.float32), o_e)
        cand = jnp.where(hit, _NEG, cand)
    out_s_ref[...] = o_s.reshape(1, 1, _BEAM)
    out_t_ref[...] = o_t.reshape(1, 1, _BEAM)
    out_e_ref[...] = o_e.reshape(1, 1, _BEAM)


def kernel(predictions, beam_scores, eos_mask):
    out_shapes = (
        jax.ShapeDtypeStruct((_BATCH, 1, _BEAM), jnp.float32),
        jax.ShapeDtypeStruct((_BATCH, 1, _BEAM), jnp.int32),
        jax.ShapeDtypeStruct((_BATCH, 1, _BEAM), jnp.float32),
    )
    grid = (_BATCH,)
    s3 = beam_scores.reshape(_BATCH, 1, _BEAM)
    e3 = eos_mask.reshape(_BATCH, 1, _BEAM)
    out = pl.pallas_call(
        _beam_block,
        grid=grid,
        in_specs=[
            pl.BlockSpec((_BEAM, _VOCAB), lambda i: (i, 0)),
            pl.BlockSpec((1, 1, _BEAM), lambda i: (i, 0, 0)),
            pl.BlockSpec((1, 1, _BEAM), lambda i: (i, 0, 0)),
        ],
        out_specs=(
            pl.BlockSpec((1, 1, _BEAM), lambda i: (i, 0, 0)),
            pl.BlockSpec((1, 1, _BEAM), lambda i: (i, 0, 0)),
            pl.BlockSpec((1, 1, _BEAM), lambda i: (i, 0, 0)),
        ),
        out_shape=out_shapes,
        scratch_shapes=[pltpu.VMEM((_BEAM, _VOCAB), jnp.float32)],
    )(predictions, s3, e3)
    return tuple(o.reshape(_BATCH, _BEAM) for o in out)


# speedup vs baseline: 2.1300x; 2.1300x over previous
import functools

import jax
import jax.numpy as jnp
from jax import lax
from jax.experimental import pallas as pl
from jax.experimental.pallas import tpu as pltpu
from jax.experimental.pallas import tpu_sc as plsc

_BATCH = 32
_BEAM = 8
_VOCAB = 100000
_EOS = 2
_NEG = -3.0e38
_L = 16
_CVEC = 50
_CELEM = _CVEC * _L
_NCHUNK = _VOCAB // _CELEM
_BIG = 2**30


def _xlane(x, op):
    lane = lax.iota(jnp.int32, _L)
    dnums = lax.GatherDimensionNumbers(offset_dims=(),
                                       collapsed_slice_dims=(0,),
                                       start_index_map=(0,))
    for s in (1, 2, 4, 8):
        y = lax.gather(x, (lane ^ s)[:, None], dimension_numbers=dnums,
                       slice_sizes=(1,),
                       mode=lax.GatherScatterMode.PROMISE_IN_BOUNDS)
        x = op(x, y)
    return x


def _sc_topk_body(pred_hbm, out_v_hbm, out_i_hbm, row_v, cmax_v, stage_v,
                  stage_i):
    wid = lax.axis_index("s") * 2 + lax.axis_index("c")
    lane = lax.iota(jnp.int32, _L)
    bigv = jnp.full((_L,), _BIG, jnp.int32)
    negv = jnp.full((_L,), _NEG, jnp.float32)

    def row_body(r, carry0):
        grow = wid * _BEAM + r
        pltpu.sync_copy(pred_hbm.at[grow], row_v)

        def p1(c, carry):
            base = c * _CELEM
            m = row_v[pl.ds(base, _L)]
            for v in range(1, _CVEC):
                m = jnp.maximum(m, row_v[pl.ds(base + v * _L, _L)])
            cmax_v[pl.ds(c * _L, _L)] = m
            return carry

        lax.fori_loop(0, _NCHUNK, p1, 0)

        vals = jnp.full((_L,), 1.0, jnp.float32)
        idxs = jnp.zeros((_L,), jnp.int32)
        for j in range(_BEAM):
            def scan_cmax(c, carry):
                g, gc = carry
                x = cmax_v[pl.ds(c * _L, _L)]
                upd = x > g
                return (jnp.where(upd, x, g),
                        jnp.where(upd, jnp.full((_L,), c, jnp.int32), gc))

            g, gc = lax.fori_loop(0, _NCHUNK, scan_cmax, (negv, bigv))
            gms = _xlane(g, jnp.maximum)
            cstar = _xlane(jnp.where(g == gms, gc, bigv), jnp.minimum)
            base_v = cstar * _CELEM

            def cpass(v, carry):
                m1, m2, fv = carry
                x = plsc.load_gather(row_v, [base_v + v * _L + lane])
                fv = jnp.minimum(
                    fv, jnp.where(x == gms, jnp.full((_L,), v, jnp.int32),
                                  bigv))
                m2 = jnp.maximum(m2, jnp.minimum(m1, x))
                m1 = jnp.maximum(m1, x)
                return m1, m2, fv

            m1, m2, fv = lax.fori_loop(0, _CVEC, cpass, (negv, negv, bigv))
            vstar = _xlane(fv, jnp.minimum)
            xv = plsc.load_gather(row_v, [base_v + vstar * _L + lane])
            lstar = _xlane(jnp.where(xv == gms, lane, bigv), jnp.minimum)
            winpos = base_v + vstar * _L + lstar

            plsc.store_scatter(row_v, [winpos], negv, mask=lane == lstar)
            newm = jnp.where(lane == lstar, m2, m1)
            plsc.store_scatter(cmax_v, [cstar * _L + lane], newm)

            sel = lane == j
            vals = jnp.where(sel, gms, vals)
            idxs = jnp.where(sel, winpos, idxs)

        stage_v[...] = vals
        stage_i[...] = idxs
        pltpu.sync_copy(stage_v, out_v_hbm.at[grow])
        pltpu.sync_copy(stage_i, out_i_hbm.at[grow])
        return carry0

    lax.fori_loop(0, _BEAM, row_body, 0)


_sc_topk = functools.partial(
    pl.kernel,
    out_type=(
        jax.ShapeDtypeStruct((_BATCH * _BEAM, _L), jnp.float32),
        jax.ShapeDtypeStruct((_BATCH * _BEAM, _L), jnp.int32),
    ),
    mesh=plsc.VectorSubcoreMesh(core_axis_name="c", subcore_axis_name="s"),
    compiler_params=pltpu.CompilerParams(needs_layout_passes=False),
    scratch_types=[
        pltpu.VMEM((_VOCAB,), jnp.float32),
        pltpu.VMEM((_NCHUNK * _L,), jnp.float32),
        pltpu.VMEM((_L,), jnp.float32),
        pltpu.VMEM((_L,), jnp.int32),
    ],
)(_sc_topk_body)


def _merge_block(v_ref, i_ref, scores_ref, eos_ref, out_s_ref, out_t_ref,
                 out_e_ref):
    top_v = v_ref[0]
    top_i = i_ref[0]
    beam_s = scores_ref[0, 0, :].reshape(_BEAM, 1)
    eos = eos_ref[0, 0, :].reshape(_BEAM, 1)

    cand = beam_s + jnp.log(top_v) * eos
    row_i = jax.lax.broadcasted_iota(jnp.int32, (_BEAM, _L), 0)
    col_i = jax.lax.broadcasted_iota(jnp.int32, (_BEAM, _L), 1)
    valid = col_i < _BEAM
    cand = jnp.where(valid, cand, _NEG)
    flat = jnp.where(valid, row_i * _BEAM + col_i, _BIG)
    pcol = jax.lax.broadcasted_iota(jnp.int32, (1, _BEAM), 1)
    eos_col = jax.lax.broadcasted_iota(jnp.int32, (_BEAM, 1), 0)

    o_s = jnp.zeros((1, _BEAM), jnp.float32)
    o_t = jnp.zeros((1, _BEAM), jnp.int32)
    o_e = jnp.zeros((1, _BEAM), jnp.float32)
    for j in range(_BEAM):
        m = jnp.max(cand)
        eq = cand == m
        sel = jnp.min(jnp.where(eq, flat, _BIG))
        hit = flat == sel
        tok = jnp.sum(jnp.where(hit, top_i, 0))
        parent = sel // _BEAM
        pe = jnp.sum(jnp.where(eos_col == parent, eos, jnp.float32(0.0)))
        slot = pcol == j
        o_s = jnp.where(slot, m, o_s)
        o_t = jnp.where(slot, tok, o_t)
        o_e = jnp.where(slot, pe * (tok != _EOS).astype(jnp.float32), o_e)
        cand = jnp.where(hit, _NEG, cand)
    out_s_ref[...] = o_s.reshape(1, 1, _BEAM)
    out_t_ref[...] = o_t.reshape(1, 1, _BEAM)
    out_e_ref[...] = o_e.reshape(1, 1, _BEAM)


def kernel(predictions, beam_scores, eos_mask):
    tv, ti = _sc_topk(predictions)
    tv = tv.reshape(_BATCH, _BEAM, _L)
    ti = ti.reshape(_BATCH, _BEAM, _L)
    s3 = beam_scores.reshape(_BATCH, 1, _BEAM)
    e3 = eos_mask.reshape(_BATCH, 1, _BEAM)
    out_shapes = (
        jax.ShapeDtypeStruct((_BATCH, 1, _BEAM), jnp.float32),
        jax.ShapeDtypeStruct((_BATCH, 1, _BEAM), jnp.int32),
        jax.ShapeDtypeStruct((_BATCH, 1, _BEAM), jnp.float32),
    )
    out = pl.pallas_call(
        _merge_block,
        grid=(_BATCH,),
        in_specs=[
            pl.BlockSpec((1, _BEAM, _L), lambda i: (i, 0, 0)),
            pl.BlockSpec((1, _BEAM, _L), lambda i: (i, 0, 0)),
            pl.BlockSpec((1, 1, _BEAM), lambda i: (i, 0, 0)),
            pl.BlockSpec((1, 1, _BEAM), lambda i: (i, 0, 0)),
        ],
        out_specs=(
            pl.BlockSpec((1, 1, _BEAM), lambda i: (i, 0, 0)),
            pl.BlockSpec((1, 1, _BEAM), lambda i: (i, 0, 0)),
            pl.BlockSpec((1, 1, _BEAM), lambda i: (i, 0, 0)),
        ),
        out_shape=out_shapes,
    )(tv, ti, s3, e3)
    return tuple(o.reshape(_BATCH, _BEAM) for o in out)


# speedup vs baseline: 2.8508x; 1.3384x over previous
import functools

import jax
import jax.numpy as jnp
from jax import lax
from jax.experimental import pallas as pl
from jax.experimental.pallas import tpu as pltpu
from jax.experimental.pallas import tpu_sc as plsc

_BATCH = 32
_BEAM = 8
_VOCAB = 100000
_EOS = 2
_NEG = -3.0e38
_L = 16
_CVEC = 50
_CELEM = _CVEC * _L
_NCHUNK = _VOCAB // _CELEM
_BIG = 2**30


def _xlane(x, op):
    lane = lax.iota(jnp.int32, _L)
    dnums = lax.GatherDimensionNumbers(offset_dims=(),
                                       collapsed_slice_dims=(0,),
                                       start_index_map=(0,))
    for s in (1, 2, 4, 8):
        y = lax.gather(x, (lane ^ s)[:, None], dimension_numbers=dnums,
                       slice_sizes=(1,),
                       mode=lax.GatherScatterMode.PROMISE_IN_BOUNDS)
        x = op(x, y)
    return x


def _sc_topk_body(pred_hbm, out_v_hbm, out_i_hbm, row_v, cmax_v, stage_v,
                  stage_i):
    wid = lax.axis_index("s") * 2 + lax.axis_index("c")
    lane = lax.iota(jnp.int32, _L)
    bigv = jnp.full((_L,), _BIG, jnp.int32)
    negv = jnp.full((_L,), _NEG, jnp.float32)

    def row_body(r, carry0):
        grow = wid * _BEAM + r
        pltpu.sync_copy(pred_hbm.at[grow], row_v)

        def p1(c, carry):
            base = c * _CELEM
            m = row_v[pl.ds(base, _L)]
            for v in range(1, _CVEC):
                m = jnp.maximum(m, row_v[pl.ds(base + v * _L, _L)])
            cmax_v[pl.ds(c * _L, _L)] = m
            return carry

        lax.fori_loop(0, _NCHUNK, p1, 0)

        vals = jnp.full((_L,), 1.0, jnp.float32)
        idxs = jnp.zeros((_L,), jnp.int32)
        for j in range(_BEAM):
            def scan_cmax(c, carry):
                g, gc = carry
                x = cmax_v[pl.ds(c * _L, _L)]
                upd = x > g
                return (jnp.where(upd, x, g),
                        jnp.where(upd, jnp.full((_L,), c, jnp.int32), gc))

            g, gc = lax.fori_loop(0, _NCHUNK, scan_cmax, (negv, bigv))
            gms = _xlane(g, jnp.maximum)
            cstar = _xlane(jnp.where(g == gms, gc, bigv), jnp.minimum)
            base_v = cstar * _CELEM

            def cpass(v, carry):
                m1, m2, fv = carry
                x = plsc.load_gather(row_v, [base_v + v * _L + lane])
                fv = jnp.minimum(
                    fv, jnp.where(x == gms, jnp.full((_L,), v, jnp.int32),
                                  bigv))
                m2 = jnp.maximum(m2, jnp.minimum(m1, x))
                m1 = jnp.maximum(m1, x)
                return m1, m2, fv

            m1, m2, fv = lax.fori_loop(0, _CVEC, cpass, (negv, negv, bigv))
            vstar = _xlane(fv, jnp.minimum)
            xv = plsc.load_gather(row_v, [base_v + vstar * _L + lane])
            lstar = _xlane(jnp.where(xv == gms, lane, bigv), jnp.minimum)
            winpos = base_v + vstar * _L + lstar

            plsc.store_scatter(row_v, [winpos], negv, mask=lane == lstar)
            newm = jnp.where(lane == lstar, m2, m1)
            plsc.store_scatter(cmax_v, [cstar * _L + lane], newm)

            sel = lane == j
            vals = jnp.where(sel, gms, vals)
            idxs = jnp.where(sel, winpos, idxs)

        stage_v[...] = vals
        stage_i[...] = idxs
        pltpu.sync_copy(stage_v, out_v_hbm.at[grow])
        pltpu.sync_copy(stage_i, out_i_hbm.at[grow])
        return carry0

    lax.fori_loop(0, _BEAM, row_body, 0)


_sc_topk = functools.partial(
    pl.kernel,
    out_type=(
        jax.ShapeDtypeStruct((_BATCH * _BEAM, _L), jnp.float32),
        jax.ShapeDtypeStruct((_BATCH * _BEAM, _L), jnp.int32),
    ),
    mesh=plsc.VectorSubcoreMesh(core_axis_name="c", subcore_axis_name="s"),
    compiler_params=pltpu.CompilerParams(needs_layout_passes=False),
    scratch_types=[
        pltpu.VMEM((_VOCAB,), jnp.float32),
        pltpu.VMEM((_NCHUNK * _L,), jnp.float32),
        pltpu.VMEM((_L,), jnp.float32),
        pltpu.VMEM((_L,), jnp.int32),
    ],
)(_sc_topk_body)


def _merge_block(v_ref, i_ref, scores_ref, eos_ref, out_s_ref, out_t_ref,
                 out_e_ref):
    v = v_ref[...]
    t = i_ref[...]
    s = scores_ref[...]
    e = eos_ref[...]

    col = jax.lax.broadcasted_iota(jnp.int32, (_BATCH, _BEAM * _L), 1)
    grp = col // _L
    sub = col % _L
    valid = sub < _BEAM
    s_exp = jnp.zeros((_BATCH, _BEAM * _L), jnp.float32)
    e_exp = jnp.zeros((_BATCH, _BEAM * _L), jnp.float32)
    for p in range(_BEAM):
        hitp = grp == p
        s_exp = jnp.where(hitp, s[:, p:p + 1], s_exp)
        e_exp = jnp.where(hitp, e[:, p:p + 1], e_exp)
    cand = s_exp + jnp.log(v) * e_exp
    cand = jnp.where(valid, cand, _NEG)
    flat = jnp.where(valid, grp * _BEAM + sub, _BIG)
    ocol = jax.lax.broadcasted_iota(jnp.int32, (_BATCH, _BEAM), 1)

    o_s = jnp.zeros((_BATCH, _BEAM), jnp.float32)
    o_t = jnp.zeros((_BATCH, _BEAM), jnp.int32)
    o_e = jnp.zeros((_BATCH, _BEAM), jnp.float32)
    for j in range(_BEAM):
        m = jnp.max(cand, axis=1, keepdims=True)
        sel = jnp.min(jnp.where(cand == m, flat, _BIG), axis=1,
                      keepdims=True)
        hit = flat == sel
        tok = jnp.sum(jnp.where(hit, t, 0), axis=1, keepdims=True)
        pe = jnp.sum(jnp.where(hit, e_exp, jnp.float32(0.0)), axis=1,
                     keepdims=True)
        slot = ocol == j
        o_s = jnp.where(slot, m, o_s)
        o_t = jnp.where(slot, tok, o_t)
        o_e = jnp.where(slot, pe * (tok != _EOS).astype(jnp.float32), o_e)
        cand = jnp.where(hit, _NEG, cand)
    out_s_ref[...] = o_s
    out_t_ref[...] = o_t
    out_e_ref[...] = o_e


def kernel(predictions, beam_scores, eos_mask):
    tv, ti = _sc_topk(predictions)
    tv = tv.reshape(_BATCH, _BEAM * _L)
    ti = ti.reshape(_BATCH, _BEAM * _L)
    out_shapes = (
        jax.ShapeDtypeStruct((_BATCH, _BEAM), jnp.float32),
        jax.ShapeDtypeStruct((_BATCH, _BEAM), jnp.int32),
        jax.ShapeDtypeStruct((_BATCH, _BEAM), jnp.float32),
    )
    return pl.pallas_call(
        _merge_block,
        out_shape=out_shapes,
    )(tv, ti, beam_scores, eos_mask)


# speedup vs baseline: 3.2401x; 1.1366x over previous
import functools

import jax
import jax.numpy as jnp
from jax import lax
from jax.experimental import pallas as pl
from jax.experimental.pallas import tpu as pltpu
from jax.experimental.pallas import tpu_sc as plsc

_BATCH = 32
_BEAM = 8
_VOCAB = 100000
_EOS = 2
_NEG = -3.0e38
_L = 16
_CVEC = 50
_CELEM = _CVEC * _L
_NCHUNK = _VOCAB // _CELEM
_BIG = 2**30


def _xlane(x, op):
    lane = lax.iota(jnp.int32, _L)
    dnums = lax.GatherDimensionNumbers(offset_dims=(),
                                       collapsed_slice_dims=(0,),
                                       start_index_map=(0,))
    for s in (1, 2, 4, 8):
        y = lax.gather(x, (lane ^ s)[:, None], dimension_numbers=dnums,
                       slice_sizes=(1,),
                       mode=lax.GatherScatterMode.PROMISE_IN_BOUNDS)
        x = op(x, y)
    return x


_CPAD = 128
_NGRP = 8
_GSZ = 16
_NSLC = 5
_SCH = _NCHUNK // _NSLC
_SELEM = _SCH * _CELEM


def _sc_topk_body(pred_hbm, out_v_hbm, out_i_hbm, row_v, cmax_v, smax_v,
                  stage_v, stage_i, sem0, sem1):
    wid = lax.axis_index("s") * 2 + lax.axis_index("c")
    lane = lax.iota(jnp.int32, _L)
    bigv = jnp.full((_L,), _BIG, jnp.int32)
    negv = jnp.full((_L,), _NEG, jnp.float32)
    sems = (sem0, sem1)

    def row_body(r, carry0):
        grow = wid * _BEAM + r

        for c in range(_NCHUNK, _CPAD):
            cmax_v[pl.ds(c * _L, _L)] = negv

        pltpu.sync_copy(pred_hbm.at[grow], row_v)

        def p1(c, carry):
            base = c * _CELEM
            m = row_v[pl.ds(base, _L)]
            for v in range(1, _CVEC):
                m = jnp.maximum(m, row_v[pl.ds(base + v * _L, _L)])
            cmax_v[pl.ds(c * _L, _L)] = m
            return carry

        lax.fori_loop(0, _NCHUNK, p1, 0)

        def build_smax(g, carry):
            m = cmax_v[pl.ds(g * _GSZ * _L, _L)]
            for k in range(1, _GSZ):
                m = jnp.maximum(m, cmax_v[pl.ds((g * _GSZ + k) * _L, _L)])
            smax_v[pl.ds(g * _L, _L)] = m
            return carry

        lax.fori_loop(0, _NGRP, build_smax, 0)

        vals = jnp.full((_L,), 1.0, jnp.float32)
        idxs = jnp.zeros((_L,), jnp.int32)
        for j in range(_BEAM):
            def scan_smax(g, carry):
                gv, gg = carry
                x = smax_v[pl.ds(g * _L, _L)]
                upd = x > gv
                return (jnp.where(upd, x, gv),
                        jnp.where(upd, jnp.full((_L,), g, jnp.int32), gg))

            gv, gg = lax.fori_loop(0, _NGRP, scan_smax, (negv, bigv))
            gms = _xlane(gv, jnp.maximum)
            ggrp = _xlane(jnp.where(gv == gms, gg, bigv), jnp.minimum)

            def scan_grp(k, fc):
                cid = ggrp * _GSZ + k
                x = plsc.load_gather(cmax_v, [cid * _L + lane])
                return jnp.minimum(fc, jnp.where(x == gms, cid, bigv))

            fc = lax.fori_loop(0, _GSZ, scan_grp, bigv)
            cstar = _xlane(fc, jnp.minimum)
            base_v = cstar * _CELEM

            def cpass(v, carry):
                m1, m2, fv = carry
                x = plsc.load_gather(row_v, [base_v + v * _L + lane])
                fv = jnp.minimum(
                    fv, jnp.where(x == gms, jnp.full((_L,), v, jnp.int32),
                                  bigv))
                m2 = jnp.maximum(m2, jnp.minimum(m1, x))
                m1 = jnp.maximum(m1, x)
                return m1, m2, fv

            m1, m2, fv = lax.fori_loop(0, _CVEC, cpass, (negv, negv, bigv))
            vstar = _xlane(fv, jnp.minimum)
            xv = plsc.load_gather(row_v, [base_v + vstar * _L + lane])
            lstar = _xlane(jnp.where(xv == gms, lane, bigv), jnp.minimum)
            winpos = base_v + vstar * _L + lstar

            plsc.store_scatter(row_v, [winpos], negv, mask=lane == lstar)
            newm = jnp.where(lane == lstar, m2, m1)
            plsc.store_scatter(cmax_v, [cstar * _L + lane], newm)

            def rebuild_smax(k, m):
                cid = ggrp * _GSZ + k
                return jnp.maximum(
                    m, plsc.load_gather(cmax_v, [cid * _L + lane]))

            ms = lax.fori_loop(0, _GSZ, rebuild_smax, negv)
            plsc.store_scatter(smax_v, [ggrp * _L + lane], ms)

            sel = lane == j
            vals = jnp.where(sel, gms, vals)
            idxs = jnp.where(sel, winpos, idxs)

        stage_v[pl.ds(r * _L, _L)] = vals
        stage_i[pl.ds(r * _L, _L)] = idxs
        return carry0

    lax.fori_loop(0, _BEAM, row_body, 0)
    pltpu.sync_copy(stage_v, out_v_hbm.at[pl.ds(wid * _BEAM * _L,
                                                _BEAM * _L)])
    pltpu.sync_copy(stage_i, out_i_hbm.at[pl.ds(wid * _BEAM * _L,
                                                _BEAM * _L)])


_sc_topk = functools.partial(
    pl.kernel,
    out_type=(
        jax.ShapeDtypeStruct((_BATCH * _BEAM * _L,), jnp.float32),
        jax.ShapeDtypeStruct((_BATCH * _BEAM * _L,), jnp.int32),
    ),
    mesh=plsc.VectorSubcoreMesh(core_axis_name="c", subcore_axis_name="s"),
    compiler_params=pltpu.CompilerParams(needs_layout_passes=False),
    scratch_types=[
        pltpu.VMEM((_VOCAB,), jnp.float32),
        pltpu.VMEM((_CPAD * _L,), jnp.float32),
        pltpu.VMEM((_NGRP * _L,), jnp.float32),
        pltpu.VMEM((_BEAM * _L,), jnp.float32),
        pltpu.VMEM((_BEAM * _L,), jnp.int32),
        pltpu.SemaphoreType.DMA,
        pltpu.SemaphoreType.DMA,
    ],
)(_sc_topk_body)


def _merge_block(v_ref, i_ref, scores_ref, eos_ref, out_s_ref, out_t_ref,
                 out_e_ref):
    v = v_ref[...]
    t = i_ref[...]
    s = scores_ref[...]
    e = eos_ref[...]

    col = jax.lax.broadcasted_iota(jnp.int32, (_BATCH, _BEAM * _L), 1)
    grp = col // _L
    sub = col % _L
    valid = sub < _BEAM
    s_exp = jnp.zeros((_BATCH, _BEAM * _L), jnp.float32)
    e_exp = jnp.zeros((_BATCH, _BEAM * _L), jnp.float32)
    for p in range(_BEAM):
        hitp = grp == p
        s_exp = jnp.where(hitp, s[:, p:p + 1], s_exp)
        e_exp = jnp.where(hitp, e[:, p:p + 1], e_exp)
    cand = s_exp + jnp.log(v) * e_exp
    cand = jnp.where(valid, cand, _NEG)
    flat = jnp.where(valid, grp * _BEAM + sub, _BIG)
    ocol = jax.lax.broadcasted_iota(jnp.int32, (_BATCH, _BEAM), 1)

    o_s = jnp.zeros((_BATCH, _BEAM), jnp.float32)
    o_t = jnp.zeros((_BATCH, _BEAM), jnp.int32)
    o_e = jnp.zeros((_BATCH, _BEAM), jnp.float32)
    for j in range(_BEAM):
        m = jnp.max(cand, axis=1, keepdims=True)
        sel = jnp.min(jnp.where(cand == m, flat, _BIG), axis=1,
                      keepdims=True)
        hit = flat == sel
        tok = jnp.sum(jnp.where(hit, t, 0), axis=1, keepdims=True)
        pe = jnp.sum(jnp.where(hit, e_exp, jnp.float32(0.0)), axis=1,
                     keepdims=True)
        slot = ocol == j
        o_s = jnp.where(slot, m, o_s)
        o_t = jnp.where(slot, tok, o_t)
        o_e = jnp.where(slot, pe * (tok != _EOS).astype(jnp.float32), o_e)
        cand = jnp.where(hit, _NEG, cand)
    out_s_ref[...] = o_s
    out_t_ref[...] = o_t
    out_e_ref[...] = o_e


def kernel(predictions, beam_scores, eos_mask):
    tv, ti = _sc_topk(predictions)
    tv = tv.reshape(_BATCH, _BEAM * _L)
    ti = ti.reshape(_BATCH, _BEAM * _L)
    out_shapes = (
        jax.ShapeDtypeStruct((_BATCH, _BEAM), jnp.float32),
        jax.ShapeDtypeStruct((_BATCH, _BEAM), jnp.int32),
        jax.ShapeDtypeStruct((_BATCH, _BEAM), jnp.float32),
    )
    return pl.pallas_call(
        _merge_block,
        out_shape=out_shapes,
    )(tv, ti, beam_scores, eos_mask)
